# baseline (speedup 1.0000x reference)
import functools

import jax
import jax.numpy as jnp
from jax import lax
from jax.experimental import pallas as pl
from jax.experimental.pallas import tpu as pltpu
from jax.experimental.pallas import tpu_sc as plsc

N_NODE = 10000
E = 320000
D_IN = 128
D_H = 16

NC = 2
NS = 16
NW = NC * NS
NPANEL = E // 128
NPW = NPANEL // NW
NEXTRA = NPANEL - NPW * NW


def _node_body(var_ref, con_ref, w1v, w1c, b1, w2, b2, w3, tv_ref, tc_ref):
    t = (jnp.dot(var_ref[:], w1v[:], precision=None)
         + jnp.dot(con_ref[:], w1c[:], precision=None) + b1[:])
    t = jnp.maximum(t, 0.0)
    t = jnp.maximum(jnp.dot(t, w2[:], precision=None) + b2[:], 0.0)
    t = jnp.dot(t, w3[:], precision=None)
    tv_ref[:] = t[:, :D_H]
    tc_ref[:] = t[:, D_H:]


def _edge_body(g4_ref, cef_ref, w1t_ref, b1_ref, w2t_ref, b2_ref, out_ref):
    pb = g4_ref.shape[0]
    n = 128 * pb
    gu = jnp.transpose(g4_ref[:, 0], (1, 0, 2)).reshape(8, n)
    gl = jnp.transpose(g4_ref[:, 1], (1, 0, 2)).reshape(8, n)
    g = jnp.concatenate([gu, gl], axis=0)
    x = jnp.dot(w1t_ref[:], cef_ref[:], precision=None) + g + b1_ref[:, 0:1]
    x = jnp.maximum(x, 0.0)
    out_ref[:] = jnp.dot(w2t_ref[:], x, precision=None) + b2_ref[:, 0:1]


def _gather_body(tv_hbm, tc_hbm, ei_hbm, g4_hbm,
                 idx_v, idx_c, idx_xv, idx_xc, rows_v, rows_c, stage,
                 sem_v, sem_c, sem_w):
    wid = lax.axis_index("s") * NC + lax.axis_index("c")
    p0 = wid * NPW
    pltpu.sync_copy(ei_hbm.at[0, pl.ds(p0, NPW)], idx_v)
    pltpu.sync_copy(ei_hbm.at[1, pl.ds(p0, NPW)], idx_c)

    iota16 = lax.iota(jnp.int32, 16)
    trv = iota16 // 8
    rv = iota16 % 8

    def fire(idxv_row, idxc_row, slot):
        pltpu.make_async_copy(tv_hbm.at[idxv_row], rows_v.at[slot],
                              sem_v.at[slot]).start()
        pltpu.make_async_copy(tc_hbm.at[idxc_row], rows_c.at[slot],
                              sem_c.at[slot]).start()

    def wait_gather(idxv_row, idxc_row, slot):
        pltpu.make_async_copy(tv_hbm.at[idxv_row], rows_v.at[slot],
                              sem_v.at[slot]).wait()
        pltpu.make_async_copy(tc_hbm.at[idxc_row], rows_c.at[slot],
                              sem_c.at[slot]).wait()

    def transpose_panel(slot, wslot):
        st = stage.at[wslot]
        for col in range(128):
            vv = rows_v[slot, col, :]
            vc = rows_c[slot, col, :]
            plsc.store_scatter(st, [trv, rv, jnp.full((16,), col, jnp.int32)],
                               vv + vc)

    def fire_write(slot, panel):
        pltpu.make_async_copy(stage.at[slot], g4_hbm.at[panel],
                              sem_w.at[slot]).start()

    def wait_write(slot, panel):
        pltpu.make_async_copy(stage.at[slot], g4_hbm.at[panel],
                              sem_w.at[slot]).wait()

    for q in range(3):
        fire(idx_v.at[q], idx_c.at[q], q)

    def body(p, _):
        slot = lax.rem(p, 4)
        wslot = lax.rem(p, 2)

        @pl.when(p + 3 < NPW)
        def _():
            fire(idx_v.at[p + 3], idx_c.at[p + 3], lax.rem(p + 3, 4))

        wait_gather(idx_v.at[p], idx_c.at[p], slot)

        @pl.when(p >= 2)
        def _():
            wait_write(wslot, p0 + p - 2)

        transpose_panel(slot, wslot)
        fire_write(wslot, p0 + p)
        return 0

    lax.fori_loop(0, NPW, body, 0)
    wait_write(0, p0 + NPW - 2)
    wait_write(1, p0 + NPW - 1)

    @pl.when(wid < NEXTRA)
    def _():
        xp = NW * NPW + wid
        pltpu.sync_copy(ei_hbm.at[0, pl.ds(xp, 1)], idx_xv)
        pltpu.sync_copy(ei_hbm.at[1, pl.ds(xp, 1)], idx_xc)
        fire(idx_xv.at[0], idx_xc.at[0], 0)
        wait_gather(idx_xv.at[0], idx_xc.at[0], 0)
        transpose_panel(0, 0)
        fire_write(0, xp)
        wait_write(0, xp)


_sc_gather = functools.partial(
    pl.kernel,
    out_type=jax.ShapeDtypeStruct((NPANEL, 2, 8, 128), jnp.float32),
    mesh=plsc.VectorSubcoreMesh(core_axis_name="c", subcore_axis_name="s"),
    scratch_types=[
        pltpu.VMEM((NPW, 128), jnp.int32),
        pltpu.VMEM((NPW, 128), jnp.int32),
        pltpu.VMEM((1, 128), jnp.int32),
        pltpu.VMEM((1, 128), jnp.int32),
        pltpu.VMEM((4, 128, D_H), jnp.float32),
        pltpu.VMEM((4, 128, D_H), jnp.float32),
        pltpu.VMEM((2, 2, 8, 128), jnp.float32),
        pltpu.SemaphoreType.DMA((4,)),
        pltpu.SemaphoreType.DMA((4,)),
        pltpu.SemaphoreType.DMA((2,)),
    ],
    compiler_params=pltpu.CompilerParams(use_tc_tiling_on_sc=False,
                                         needs_layout_passes=False),
)(_gather_body)


def kernel(var_f, con_f, combined_edge_f, edge_index_var_con,
           vW1, vb1, vW2, vb2, cW1, cb1, cW2, cb2, eW1, eb1, eW2, eb2):
    f32 = jnp.float32
    z = jnp.zeros((D_IN, D_H), f32)
    w1v = jnp.concatenate([vW1, z], axis=1)
    w1c = jnp.concatenate([z, cW1], axis=1)
    b1 = jnp.concatenate([vb1, cb1]).reshape(1, 2 * D_H)
    z2 = jnp.zeros((D_H, D_H), f32)
    w2 = jnp.block([[vW2, z2], [z2, cW2]])
    b2 = jnp.concatenate([vb2, cb2]).reshape(1, 2 * D_H)
    w3 = jnp.block([[eW1[D_H:2 * D_H], z2], [z2, eW1[2 * D_H:]]])

    tv, tc = pl.pallas_call(
        _node_body,
        out_shape=(
            jax.ShapeDtypeStruct((N_NODE, D_H), f32),
            jax.ShapeDtypeStruct((N_NODE, D_H), f32),
        ),
    )(var_f, con_f, w1v, w1c, b1, w2, b2, w3)

    ei = edge_index_var_con.astype(jnp.int32).reshape(2, NPANEL, 128)
    g4 = _sc_gather(tv, tc, ei)

    cefT = combined_edge_f.T
    w1t = eW1[:D_H].T
    w2t = eW2.T
    b1e = jnp.tile(eb1.reshape(D_H, 1), (1, 128))
    b2e = jnp.tile(eb2.reshape(D_H, 1), (1, 128))

    PB = 125
    outT = pl.pallas_call(
        _edge_body,
        grid=(NPANEL // PB,),
        in_specs=[
            pl.BlockSpec((PB, 2, 8, 128), lambda i: (i, 0, 0, 0)),
            pl.BlockSpec((D_H, 128 * PB), lambda i: (0, i)),
            pl.BlockSpec((D_H, D_H), lambda i: (0, 0)),
            pl.BlockSpec((D_H, 128), lambda i: (0, 0)),
            pl.BlockSpec((D_H, D_H), lambda i: (0, 0)),
            pl.BlockSpec((D_H, 128), lambda i: (0, 0)),
        ],
        out_specs=pl.BlockSpec((D_H, 128 * PB), lambda i: (0, i)),
        out_shape=jax.ShapeDtypeStruct((D_H, E), f32),
    )(g4, cefT, w1t, b1e, w2t, b2e)
    return outT.T


# speedup vs baseline: 1.5760x; 1.5760x over previous
import functools

import jax
import jax.numpy as jnp
from jax import lax
from jax.experimental import pallas as pl
from jax.experimental.pallas import tpu as pltpu
from jax.experimental.pallas import tpu_sc as plsc

N_NODE = 10000
E = 320000
D_IN = 128
D_H = 16

NC = 2
NS = 16
NW = NC * NS
NPANEL = E // 128
NPW = NPANEL // NW
NEXTRA = NPANEL - NPW * NW


def _node_body(var_ref, con_ref, w1v, w1c, b1, w2, b2, w3, tv_ref, tc_ref):
    t = (jnp.dot(var_ref[:], w1v[:], precision=None)
         + jnp.dot(con_ref[:], w1c[:], precision=None) + b1[:])
    t = jnp.maximum(t, 0.0)
    t = jnp.maximum(jnp.dot(t, w2[:], precision=None) + b2[:], 0.0)
    t = jnp.dot(t, w3[:], precision=None)
    tv_ref[:] = t[:, :D_H]
    tc_ref[:] = t[:, D_H:]


def _edge_body(g4_ref, cef_ref, w1t_ref, b1_ref, w2t_ref, b2_ref, out_ref):
    pb = g4_ref.shape[0]
    n = 128 * pb
    gu = jnp.transpose(g4_ref[:, 0], (1, 0, 2)).reshape(8, n)
    gl = jnp.transpose(g4_ref[:, 1], (1, 0, 2)).reshape(8, n)
    g = jnp.concatenate([gu, gl], axis=0)
    x = jnp.dot(w1t_ref[:], cef_ref[:], precision=None) + g + b1_ref[:, 0:1]
    x = jnp.maximum(x, 0.0)
    out_ref[:] = jnp.dot(w2t_ref[:], x, precision=None) + b2_ref[:, 0:1]


def _gather_body(tv_hbm, tc_hbm, ei_hbm, g4_hbm,
                 idx_v, idx_c, idx_xv, idx_xc, rows_v, rows_c, stage,
                 sem_v, sem_c, sem_w):
    wid = lax.axis_index("s") * NC + lax.axis_index("c")
    p0 = wid * NPW
    pltpu.sync_copy(ei_hbm.at[0, pl.ds(p0, NPW)], idx_v)
    pltpu.sync_copy(ei_hbm.at[1, pl.ds(p0, NPW)], idx_c)

    iota16 = lax.iota(jnp.int32, 16)
    trv = iota16 // 8
    rv = iota16 % 8

    def fire(idxv_row, idxc_row, slot):
        pltpu.make_async_copy(tv_hbm.at[idxv_row], rows_v.at[slot],
                              sem_v.at[slot]).start()
        pltpu.make_async_copy(tc_hbm.at[idxc_row], rows_c.at[slot],
                              sem_c.at[slot]).start()

    def wait_gather(idxv_row, idxc_row, slot):
        pltpu.make_async_copy(tv_hbm.at[idxv_row], rows_v.at[slot],
                              sem_v.at[slot]).wait()
        pltpu.make_async_copy(tc_hbm.at[idxc_row], rows_c.at[slot],
                              sem_c.at[slot]).wait()

    def transpose_panel(slot, wslot):
        st = stage.at[wslot]

        def col_body(col, _):
            vv = rows_v[slot, col, :]
            vc = rows_c[slot, col, :]
            plsc.store_scatter(st, [trv, rv, jnp.full((16,), col, jnp.int32)],
                               vv + vc)
            return 0
        lax.fori_loop(0, 128, col_body, 0, unroll=4)

    def fire_write(slot, panel):
        pltpu.make_async_copy(stage.at[slot, :, :, pl.ds(0, 128)],
                              g4_hbm.at[panel], sem_w.at[slot]).start()

    def wait_write(slot, panel):
        pltpu.make_async_copy(stage.at[slot, :, :, pl.ds(0, 128)],
                              g4_hbm.at[panel], sem_w.at[slot]).wait()

    for q in range(3):
        fire(idx_v.at[q], idx_c.at[q], q)

    def body(p, _):
        slot = lax.rem(p, 4)
        wslot = lax.rem(p, 2)

        @pl.when(p + 3 < NPW)
        def _():
            fire(idx_v.at[p + 3], idx_c.at[p + 3], lax.rem(p + 3, 4))

        wait_gather(idx_v.at[p], idx_c.at[p], slot)

        @pl.when(p >= 2)
        def _():
            wait_write(wslot, p0 + p - 2)

        transpose_panel(slot, wslot)
        fire_write(wslot, p0 + p)
        return 0

    lax.fori_loop(0, NPW, body, 0)
    wait_write(0, p0 + NPW - 2)
    wait_write(1, p0 + NPW - 1)

    @pl.when(wid < NEXTRA)
    def _():
        xp = NW * NPW + wid
        pltpu.sync_copy(ei_hbm.at[0, pl.ds(xp, 1)], idx_xv)
        pltpu.sync_copy(ei_hbm.at[1, pl.ds(xp, 1)], idx_xc)
        fire(idx_xv.at[0], idx_xc.at[0], 0)
        wait_gather(idx_xv.at[0], idx_xc.at[0], 0)
        transpose_panel(0, 0)
        fire_write(0, xp)
        wait_write(0, xp)


_sc_gather = functools.partial(
    pl.kernel,
    out_type=jax.ShapeDtypeStruct((NPANEL, 2, 8, 128), jnp.float32),
    mesh=plsc.VectorSubcoreMesh(core_axis_name="c", subcore_axis_name="s"),
    scratch_types=[
        pltpu.VMEM((NPW, 128), jnp.int32),
        pltpu.VMEM((NPW, 128), jnp.int32),
        pltpu.VMEM((1, 128), jnp.int32),
        pltpu.VMEM((1, 128), jnp.int32),
        pltpu.VMEM((4, 128, D_H), jnp.float32),
        pltpu.VMEM((4, 128, D_H), jnp.float32),
        pltpu.VMEM((2, 2, 8, 129), jnp.float32),
        pltpu.SemaphoreType.DMA((4,)),
        pltpu.SemaphoreType.DMA((4,)),
        pltpu.SemaphoreType.DMA((2,)),
    ],
    compiler_params=pltpu.CompilerParams(use_tc_tiling_on_sc=False,
                                         needs_layout_passes=False),
)(_gather_body)


def kernel(var_f, con_f, combined_edge_f, edge_index_var_con,
           vW1, vb1, vW2, vb2, cW1, cb1, cW2, cb2, eW1, eb1, eW2, eb2):
    f32 = jnp.float32
    z = jnp.zeros((D_IN, D_H), f32)
    w1v = jnp.concatenate([vW1, z], axis=1)
    w1c = jnp.concatenate([z, cW1], axis=1)
    b1 = jnp.concatenate([vb1, cb1]).reshape(1, 2 * D_H)
    z2 = jnp.zeros((D_H, D_H), f32)
    w2 = jnp.block([[vW2, z2], [z2, cW2]])
    b2 = jnp.concatenate([vb2, cb2]).reshape(1, 2 * D_H)
    w3 = jnp.block([[eW1[D_H:2 * D_H], z2], [z2, eW1[2 * D_H:]]])

    tv, tc = pl.pallas_call(
        _node_body,
        out_shape=(
            jax.ShapeDtypeStruct((N_NODE, D_H), f32),
            jax.ShapeDtypeStruct((N_NODE, D_H), f32),
        ),
    )(var_f, con_f, w1v, w1c, b1, w2, b2, w3)

    ei = edge_index_var_con.astype(jnp.int32).reshape(2, NPANEL, 128)
    g4 = _sc_gather(tv, tc, ei)

    cefT = combined_edge_f.T
    w1t = eW1[:D_H].T
    w2t = eW2.T
    b1e = jnp.tile(eb1.reshape(D_H, 1), (1, 128))
    b2e = jnp.tile(eb2.reshape(D_H, 1), (1, 128))

    PB = 125
    outT = pl.pallas_call(
        _edge_body,
        grid=(NPANEL // PB,),
        in_specs=[
            pl.BlockSpec((PB, 2, 8, 128), lambda i: (i, 0, 0, 0)),
            pl.BlockSpec((D_H, 128 * PB), lambda i: (0, i)),
            pl.BlockSpec((D_H, D_H), lambda i: (0, 0)),
            pl.BlockSpec((D_H, 128), lambda i: (0, 0)),
            pl.BlockSpec((D_H, D_H), lambda i: (0, 0)),
            pl.BlockSpec((D_H, 128), lambda i: (0, 0)),
        ],
        out_specs=pl.BlockSpec((D_H, 128 * PB), lambda i: (0, i)),
        out_shape=jax.ShapeDtypeStruct((D_H, E), f32),
    )(g4, cefT, w1t, b1e, w2t, b2e)
    return outT.T
